# baseline (speedup 1.0000x reference)
import functools

import jax
import jax.numpy as jnp
from jax import lax
from jax.experimental import pallas as pl
from jax.experimental.pallas import tpu as pltpu
from jax.experimental.pallas import tpu_sc as plsc

N = 10000
D = 128
E = 320000
BN_EPS = 1e-5

NC = 2
NS = 16
NT = NC * NS
K = 128
CHUNKS = 80
GB = 8
NG = CHUNKS // GB
EPT = K * CHUNKS
EPAD = EPT * NT
NPAD = 10112
ROWS_PER_SUB = NPAD // NS
DEGPAD = 10240
DEG_PER_SUB = DEGPAD // NS

_mesh = lambda: plsc.VectorSubcoreMesh(core_axis_name="c", subcore_axis_name="s")


@functools.partial(
    pl.kernel,
    mesh=_mesh(),
    out_type=jax.ShapeDtypeStruct((NC, DEGPAD), jnp.float32),
    scratch_types=[
        pltpu.VMEM((CHUNKS, K), jnp.int32),
        pltpu.VMEM((K,), jnp.float32),
        pltpu.VMEM((DEG_PER_SUB,), jnp.float32),
        pltpu.VMEM_SHARED((DEGPAD,), jnp.float32),
    ],
)
def _deg_kernel(dst_hbm, out_hbm, didx, ones_v, zeros_v, dacc):
    c = lax.axis_index("c")
    s = lax.axis_index("s")
    w = s * NC + c
    pltpu.sync_copy(dst_hbm.at[w], didx)
    for i in range(K // 16):
        ones_v[pl.ds(i * 16, 16)] = jnp.ones((16,), jnp.float32)
    for i in range(DEG_PER_SUB // 16):
        zeros_v[pl.ds(i * 16, 16)] = jnp.zeros((16,), jnp.float32)
    pltpu.sync_copy(zeros_v, dacc.at[pl.ds(s * DEG_PER_SUB, DEG_PER_SUB)])
    plsc.subcore_barrier()

    def body(j, carry):
        pltpu.sync_copy(ones_v, dacc.at[didx.at[j]], add=True)
        return carry

    lax.fori_loop(0, CHUNKS, body, 0)
    plsc.subcore_barrier()
    pltpu.sync_copy(dacc.at[pl.ds(s * DEG_PER_SUB, DEG_PER_SUB)],
                    out_hbm.at[c, pl.ds(s * DEG_PER_SUB, DEG_PER_SUB)])


@functools.partial(
    pl.kernel,
    mesh=_mesh(),
    out_type=jax.ShapeDtypeStruct((NC, NPAD, D), jnp.float32),
    scratch_types=[
        pltpu.VMEM((2, GB, K), jnp.int32),
        pltpu.VMEM((2, GB, K), jnp.int32),
        pltpu.VMEM((2, K, D), jnp.float32),
        pltpu.VMEM_SHARED((NPAD, D), jnp.float32),
        pltpu.SemaphoreType.DMA((2,)),
        pltpu.SemaphoreType.DMA((2,)),
        pltpu.SemaphoreType.DMA((2,)),
    ],
)
def _agg_kernel(hs_hbm, src_hbm, dst_hbm, zeros_hbm, out_hbm,
                sidx, didx, rows, acc, semg, semi, sems):
    c = lax.axis_index("c")
    s = lax.axis_index("s")
    w = s * NC + c
    pltpu.sync_copy(zeros_hbm, acc.at[pl.ds(s * ROWS_PER_SUB, ROWS_PER_SUB)])
    plsc.subcore_barrier()

    pltpu.sync_copy(src_hbm.at[w, pl.ds(0, GB)], sidx.at[0])
    pltpu.sync_copy(dst_hbm.at[w, pl.ds(0, GB)], didx.at[0])
    pltpu.async_copy(hs_hbm.at[sidx.at[0, 0]], rows.at[0], semg.at[0])

    def group(g, carry):
        gb = lax.rem(g, 2)
        ngb = lax.rem(g + 1, 2)

        @pl.when(g + 1 < NG)
        def _():
            pltpu.async_copy(src_hbm.at[w, pl.ds((g + 1) * GB, GB)],
                             sidx.at[ngb], semi.at[0])
            pltpu.async_copy(dst_hbm.at[w, pl.ds((g + 1) * GB, GB)],
                             didx.at[ngb], semi.at[1])

        handles = [None] * GB
        for jj in range(GB):
            b = jj % 2
            nb = (jj + 1) % 2
            pltpu.make_async_copy(hs_hbm.at[sidx.at[gb, jj]],
                                  rows.at[b], semg.at[b]).wait()
            if jj >= 1:
                handles[jj - 1].wait()
            if jj < GB - 1:
                pltpu.async_copy(hs_hbm.at[sidx.at[gb, jj + 1]],
                                 rows.at[nb], semg.at[nb])
            else:
                @pl.when(g + 1 < NG)
                def _():
                    pltpu.make_async_copy(
                        src_hbm.at[w, pl.ds((g + 1) * GB, GB)],
                        sidx.at[ngb], semi.at[0]).wait()
                    pltpu.make_async_copy(
                        dst_hbm.at[w, pl.ds((g + 1) * GB, GB)],
                        didx.at[ngb], semi.at[1]).wait()
                    pltpu.async_copy(hs_hbm.at[sidx.at[ngb, 0]],
                                     rows.at[nb], semg.at[nb])
            handles[jj] = pltpu.async_copy(rows.at[b], acc.at[didx.at[gb, jj]],
                                           sems.at[b], add=True)
        handles[GB - 1].wait()
        return carry

    lax.fori_loop(0, NG, group, 0)
    plsc.subcore_barrier()
    pltpu.sync_copy(acc.at[pl.ds(s * ROWS_PER_SUB, ROWS_PER_SUB)],
                    out_hbm.at[c, pl.ds(s * ROWS_PER_SUB, ROWS_PER_SUB)])


def _tc1_body(degp_ref, x_ref, w1_ref, hs1_ref, dinv_ref):
    deg = degp_ref[0, :N, :] + degp_ref[1, :N, :] + 1.0
    dinv = lax.rsqrt(deg)
    h = jnp.dot(x_ref[...], w1_ref[...], preferred_element_type=jnp.float32)
    hs1_ref[...] = h * dinv
    dinv_ref[...] = dinv


def _tc2_body(p_ref, hs1_ref, dinv_ref, g_ref, bt_ref, b1_ref, w2_ref, hs2_ref):
    agg = (p_ref[0, :N, :] + p_ref[1, :N, :] + hs1_ref[...]) * dinv_ref[...]
    t = agg + b1_ref[...]
    t = g_ref[...] * t * (1.0 / (1.0 + BN_EPS) ** 0.5) + bt_ref[...]
    scale = 1.0507009873554804934193349852946
    alpha = 1.6732632423543772848170429916717
    t = scale * jnp.where(t > 0, t, alpha * (jnp.exp(t) - 1.0))
    h2 = jnp.dot(t, w2_ref[...], preferred_element_type=jnp.float32)
    hs2_ref[...] = h2 * dinv_ref[...]


def _tc3_body(p_ref, hs2_ref, dinv_ref, b2_ref, out_ref):
    out_ref[...] = ((p_ref[0, :N, :] + p_ref[1, :N, :] + hs2_ref[...])
                    * dinv_ref[...] + b2_ref[...])


_tc1 = pl.pallas_call(
    _tc1_body,
    out_shape=[jax.ShapeDtypeStruct((N, D), jnp.float32),
               jax.ShapeDtypeStruct((N, 1), jnp.float32)],
)

_tc2 = pl.pallas_call(
    _tc2_body,
    out_shape=jax.ShapeDtypeStruct((N, D), jnp.float32),
)

_tc3 = pl.pallas_call(
    _tc3_body,
    out_shape=jax.ShapeDtypeStruct((N, D), jnp.float32),
)


def kernel(x, edge_index, W1, b1, gamma1, beta1, W2, b2):
    pad = EPAD - E
    pad_dst = N + (jnp.arange(pad, dtype=jnp.int32) % (NPAD - N))
    pad_src = jnp.arange(pad, dtype=jnp.int32) % N
    src = jnp.concatenate([edge_index[0], pad_src])
    dst = jnp.concatenate([edge_index[1], pad_dst])
    srcp = src.reshape(NT, CHUNKS, K)
    dstp = dst.reshape(NT, CHUNKS, K)
    zeros_rows = jnp.zeros((ROWS_PER_SUB, D), jnp.float32)

    degp = _deg_kernel(dstp)
    degp3 = degp.reshape(NC, DEGPAD, 1)

    hs1, dinv = _tc1(degp3, x, W1)
    p1 = _agg_kernel(hs1, srcp, dstp, zeros_rows)
    hs2 = _tc2(p1, hs1, dinv, gamma1.reshape(1, D), beta1.reshape(1, D),
               b1.reshape(1, D), W2)
    p2 = _agg_kernel(hs2, srcp, dstp, zeros_rows)
    out = _tc3(p2, hs2, dinv, b2.reshape(1, D))
    return out


# speedup vs baseline: 1.1637x; 1.1637x over previous
import functools

import jax
import jax.numpy as jnp
from jax import lax
from jax.experimental import pallas as pl
from jax.experimental.pallas import tpu as pltpu
from jax.experimental.pallas import tpu_sc as plsc

N = 10000
D = 128
E = 320000
BN_EPS = 1e-5

NC = 2
NS = 16
NT = NC * NS
K = 128
CHUNKS = 80
GB = 16
NG = CHUNKS // GB
EPT = K * CHUNKS
EPAD = EPT * NT
NPAD = 10112
ROWS_PER_SUB = NPAD // NS
DEGPAD = 10240
DEG_PER_SUB = DEGPAD // NS

_mesh = lambda: plsc.VectorSubcoreMesh(core_axis_name="c", subcore_axis_name="s")


@functools.partial(
    pl.kernel,
    mesh=_mesh(),
    out_type=jax.ShapeDtypeStruct((NC, DEGPAD), jnp.float32),
    scratch_types=[
        pltpu.VMEM((CHUNKS, K), jnp.int32),
        pltpu.VMEM((K,), jnp.float32),
        pltpu.VMEM((DEG_PER_SUB,), jnp.float32),
        pltpu.VMEM_SHARED((DEGPAD,), jnp.float32),
        pltpu.SemaphoreType.DMA,
    ],
)
def _deg_kernel(dst_hbm, out_hbm, didx, ones_v, zeros_v, dacc, sem):
    c = lax.axis_index("c")
    s = lax.axis_index("s")
    w = s * NC + c
    pltpu.sync_copy(dst_hbm.at[w], didx)
    for i in range(K // 16):
        ones_v[pl.ds(i * 16, 16)] = jnp.ones((16,), jnp.float32)
    for i in range(DEG_PER_SUB // 16):
        zeros_v[pl.ds(i * 16, 16)] = jnp.zeros((16,), jnp.float32)
    pltpu.sync_copy(zeros_v, dacc.at[pl.ds(s * DEG_PER_SUB, DEG_PER_SUB)])
    plsc.subcore_barrier()

    def body(j, carry):
        pltpu.async_copy(ones_v, dacc.at[didx.at[j]], sem, add=True)
        return carry

    lax.fori_loop(0, CHUNKS, body, 0)

    def drain(j, carry):
        pltpu.make_async_copy(ones_v, dacc.at[didx.at[j]], sem).wait()
        return carry

    lax.fori_loop(0, CHUNKS, drain, 0)
    plsc.subcore_barrier()
    pltpu.sync_copy(dacc.at[pl.ds(s * DEG_PER_SUB, DEG_PER_SUB)],
                    out_hbm.at[c, pl.ds(s * DEG_PER_SUB, DEG_PER_SUB)])


@functools.partial(
    pl.kernel,
    mesh=_mesh(),
    out_type=jax.ShapeDtypeStruct((NC, NPAD, D), jnp.float32),
    scratch_types=[
        pltpu.VMEM((2, GB, K), jnp.int32),
        pltpu.VMEM((2, GB, K), jnp.int32),
        pltpu.VMEM((2, K, D), jnp.float32),
        pltpu.VMEM_SHARED((NPAD, D), jnp.float32),
        pltpu.SemaphoreType.DMA((2,)),
        pltpu.SemaphoreType.DMA((2,)),
    ],
)
def _agg_kernel(hs_hbm, src_hbm, dst_hbm, zeros_hbm, out_hbm,
                sidx, didx, rows, acc, semg, semi):
    c = lax.axis_index("c")
    s = lax.axis_index("s")
    w = s * NC + c
    pltpu.sync_copy(zeros_hbm, acc.at[pl.ds(s * ROWS_PER_SUB, ROWS_PER_SUB)])
    plsc.subcore_barrier()

    pltpu.sync_copy(src_hbm.at[w, pl.ds(0, GB)], sidx.at[0])
    pltpu.sync_copy(dst_hbm.at[w, pl.ds(0, GB)], didx.at[0])
    pltpu.async_copy(hs_hbm.at[sidx.at[0, 0]], rows.at[0], semg.at[0])

    def group(g, carry):
        gb = lax.rem(g, 2)
        ngb = lax.rem(g + 1, 2)

        @pl.when(g + 1 < NG)
        def _():
            pltpu.async_copy(src_hbm.at[w, pl.ds((g + 1) * GB, GB)],
                             sidx.at[ngb], semi.at[0])
            pltpu.async_copy(dst_hbm.at[w, pl.ds((g + 1) * GB, GB)],
                             didx.at[ngb], semi.at[1])

        for jj in range(GB):
            b = jj % 2
            nb = (jj + 1) % 2
            if jj < GB - 1:
                pltpu.async_copy(hs_hbm.at[sidx.at[gb, jj + 1]],
                                 rows.at[nb], semg.at[nb])
            else:
                @pl.when(g + 1 < NG)
                def _():
                    pltpu.make_async_copy(
                        src_hbm.at[w, pl.ds((g + 1) * GB, GB)],
                        sidx.at[ngb], semi.at[0]).wait()
                    pltpu.make_async_copy(
                        dst_hbm.at[w, pl.ds((g + 1) * GB, GB)],
                        didx.at[ngb], semi.at[1]).wait()
                    pltpu.async_copy(hs_hbm.at[sidx.at[ngb, 0]],
                                     rows.at[nb], semg.at[nb])
            pltpu.make_async_copy(hs_hbm.at[sidx.at[gb, jj]],
                                  rows.at[b], semg.at[b]).wait()
            pltpu.sync_copy(rows.at[b], acc.at[didx.at[gb, jj]], add=True)
        return carry

    lax.fori_loop(0, NG, group, 0)
    plsc.subcore_barrier()
    pltpu.sync_copy(acc.at[pl.ds(s * ROWS_PER_SUB, ROWS_PER_SUB)],
                    out_hbm.at[c, pl.ds(s * ROWS_PER_SUB, ROWS_PER_SUB)])


def _tc1_body(degp_ref, x_ref, w1_ref, hs1_ref, dinv_ref):
    deg = degp_ref[0, :N, :] + degp_ref[1, :N, :] + 1.0
    dinv = lax.rsqrt(deg)
    h = jnp.dot(x_ref[...], w1_ref[...], preferred_element_type=jnp.float32)
    hs1_ref[...] = h * dinv
    dinv_ref[...] = dinv


def _tc2_body(p_ref, hs1_ref, dinv_ref, g_ref, bt_ref, b1_ref, w2_ref, hs2_ref):
    agg = (p_ref[0, :N, :] + p_ref[1, :N, :] + hs1_ref[...]) * dinv_ref[...]
    t = agg + b1_ref[...]
    t = g_ref[...] * t * (1.0 / (1.0 + BN_EPS) ** 0.5) + bt_ref[...]
    scale = 1.0507009873554804934193349852946
    alpha = 1.6732632423543772848170429916717
    t = scale * jnp.where(t > 0, t, alpha * (jnp.exp(t) - 1.0))
    h2 = jnp.dot(t, w2_ref[...], preferred_element_type=jnp.float32)
    hs2_ref[...] = h2 * dinv_ref[...]


def _tc3_body(p_ref, hs2_ref, dinv_ref, b2_ref, out_ref):
    out_ref[...] = ((p_ref[0, :N, :] + p_ref[1, :N, :] + hs2_ref[...])
                    * dinv_ref[...] + b2_ref[...])


_tc1 = pl.pallas_call(
    _tc1_body,
    out_shape=[jax.ShapeDtypeStruct((N, D), jnp.float32),
               jax.ShapeDtypeStruct((N, 1), jnp.float32)],
)

_tc2 = pl.pallas_call(
    _tc2_body,
    out_shape=jax.ShapeDtypeStruct((N, D), jnp.float32),
)

_tc3 = pl.pallas_call(
    _tc3_body,
    out_shape=jax.ShapeDtypeStruct((N, D), jnp.float32),
)


def kernel(x, edge_index, W1, b1, gamma1, beta1, W2, b2):
    pad = EPAD - E
    pad_dst = N + (jnp.arange(pad, dtype=jnp.int32) % (NPAD - N))
    pad_src = jnp.arange(pad, dtype=jnp.int32) % N
    src = jnp.concatenate([edge_index[0], pad_src])
    dst = jnp.concatenate([edge_index[1], pad_dst])
    srcp = src.reshape(NT, CHUNKS, K)
    dstp = dst.reshape(NT, CHUNKS, K)
    zeros_rows = jnp.zeros((ROWS_PER_SUB, D), jnp.float32)

    degp = _deg_kernel(dstp)
    degp3 = degp.reshape(NC, DEGPAD, 1)

    hs1, dinv = _tc1(degp3, x, W1)
    p1 = _agg_kernel(hs1, srcp, dstp, zeros_rows)
    hs2 = _tc2(p1, hs1, dinv, gamma1.reshape(1, D), beta1.reshape(1, D),
               b1.reshape(1, D), W2)
    p2 = _agg_kernel(hs2, srcp, dstp, zeros_rows)
    out = _tc3(p2, hs2, dinv, b2.reshape(1, D))
    return out


# speedup vs baseline: 1.1725x; 1.0075x over previous
import functools

import jax
import jax.numpy as jnp
from jax import lax
from jax.experimental import pallas as pl
from jax.experimental.pallas import tpu as pltpu
from jax.experimental.pallas import tpu_sc as plsc

N = 10000
D = 128
E = 320000
BN_EPS = 1e-5

NC = 2
NS = 16
NT = NC * NS
K = 128
CHUNKS = 80
GB = 16
NG = CHUNKS // GB
EPT = K * CHUNKS
EPAD = EPT * NT
NPAD = 10112
ROWS_PER_SUB = NPAD // NS
DEGPAD = 10240
DEG_PER_SUB = DEGPAD // NS

_mesh = lambda: plsc.VectorSubcoreMesh(core_axis_name="c", subcore_axis_name="s")


@functools.partial(
    pl.kernel,
    mesh=_mesh(),
    out_type=jax.ShapeDtypeStruct((NC, DEGPAD), jnp.float32),
    scratch_types=[
        pltpu.VMEM((CHUNKS, K), jnp.int32),
        pltpu.VMEM((K,), jnp.float32),
        pltpu.VMEM((DEG_PER_SUB,), jnp.float32),
        pltpu.VMEM_SHARED((DEGPAD,), jnp.float32),
        pltpu.SemaphoreType.DMA,
    ],
)
def _deg_kernel(dst_hbm, out_hbm, didx, ones_v, zeros_v, dacc, sem):
    c = lax.axis_index("c")
    s = lax.axis_index("s")
    w = s * NC + c
    pltpu.sync_copy(dst_hbm.at[w], didx)
    for i in range(K // 16):
        ones_v[pl.ds(i * 16, 16)] = jnp.ones((16,), jnp.float32)
    for i in range(DEG_PER_SUB // 16):
        zeros_v[pl.ds(i * 16, 16)] = jnp.zeros((16,), jnp.float32)
    pltpu.sync_copy(zeros_v, dacc.at[pl.ds(s * DEG_PER_SUB, DEG_PER_SUB)])
    plsc.subcore_barrier()

    def body(j, carry):
        pltpu.async_copy(ones_v, dacc.at[didx.at[j]], sem, add=True)
        return carry

    lax.fori_loop(0, CHUNKS, body, 0)

    def drain(j, carry):
        pltpu.make_async_copy(ones_v, dacc.at[didx.at[j]], sem).wait()
        return carry

    lax.fori_loop(0, CHUNKS, drain, 0)
    plsc.subcore_barrier()
    pltpu.sync_copy(dacc.at[pl.ds(s * DEG_PER_SUB, DEG_PER_SUB)],
                    out_hbm.at[c, pl.ds(s * DEG_PER_SUB, DEG_PER_SUB)])


@functools.partial(
    pl.kernel,
    mesh=_mesh(),
    out_type=jax.ShapeDtypeStruct((NC, NPAD, D), jnp.float32),
    scratch_types=[
        pltpu.VMEM((2, GB, K), jnp.int32),
        pltpu.VMEM((2, GB, K), jnp.int32),
        pltpu.VMEM((2, K, D), jnp.float32),
        pltpu.VMEM_SHARED((NPAD, D), jnp.float32),
        pltpu.SemaphoreType.DMA((2,)),
        pltpu.SemaphoreType.DMA((2,)),
    ],
)
def _agg_kernel(hs_hbm, src_hbm, dst_hbm, zeros_hbm, out_hbm,
                sidx, didx, rows, acc, semg, semi):
    c = lax.axis_index("c")
    s = lax.axis_index("s")
    w = s * NC + c
    pltpu.sync_copy(zeros_hbm, acc.at[pl.ds(s * ROWS_PER_SUB, ROWS_PER_SUB)])
    plsc.subcore_barrier()

    pltpu.sync_copy(src_hbm.at[w, pl.ds(0, GB)], sidx.at[0])
    pltpu.sync_copy(dst_hbm.at[w, pl.ds(0, GB)], didx.at[0])
    pltpu.async_copy(hs_hbm.at[sidx.at[0, 0]], rows.at[0], semg.at[0])

    def group(g, carry):
        gb = lax.rem(g, 2)
        ngb = lax.rem(g + 1, 2)

        @pl.when(g + 1 < NG)
        def _():
            pltpu.async_copy(src_hbm.at[w, pl.ds((g + 1) * GB, GB)],
                             sidx.at[ngb], semi.at[0])
            pltpu.async_copy(dst_hbm.at[w, pl.ds((g + 1) * GB, GB)],
                             didx.at[ngb], semi.at[1])

        for jj in range(GB):
            b = jj % 2
            nb = (jj + 1) % 2
            if jj < GB - 1:
                pltpu.async_copy(hs_hbm.at[sidx.at[gb, jj + 1]],
                                 rows.at[nb], semg.at[nb])
            else:
                @pl.when(g + 1 < NG)
                def _():
                    pltpu.make_async_copy(
                        src_hbm.at[w, pl.ds((g + 1) * GB, GB)],
                        sidx.at[ngb], semi.at[0]).wait()
                    pltpu.make_async_copy(
                        dst_hbm.at[w, pl.ds((g + 1) * GB, GB)],
                        didx.at[ngb], semi.at[1]).wait()
                    pltpu.async_copy(hs_hbm.at[sidx.at[ngb, 0]],
                                     rows.at[nb], semg.at[nb])
            pltpu.make_async_copy(hs_hbm.at[sidx.at[gb, jj]],
                                  rows.at[b], semg.at[b]).wait()
            pltpu.sync_copy(rows.at[b], acc.at[didx.at[gb, jj]], add=True)
        return carry

    lax.fori_loop(0, NG, group, 0)
    plsc.subcore_barrier()
    pltpu.sync_copy(acc.at[pl.ds(s * ROWS_PER_SUB, ROWS_PER_SUB)],
                    out_hbm.at[c, pl.ds(s * ROWS_PER_SUB, ROWS_PER_SUB)])


def _tc1a_body(x_ref, w1_ref, h1_ref):
    h1_ref[...] = jnp.dot(x_ref[...], w1_ref[...],
                          preferred_element_type=jnp.float32)


def _tc1b_body(degp_ref, h1_ref, hs1_ref, dinv_ref):
    deg = degp_ref[0, :N, :] + degp_ref[1, :N, :] + 1.0
    dinv = lax.rsqrt(deg)
    hs1_ref[...] = h1_ref[...] * dinv
    dinv_ref[...] = dinv


def _tc2_body(p_ref, hs1_ref, dinv_ref, g_ref, bt_ref, b1_ref, w2_ref, hs2_ref):
    agg = (p_ref[0, :N, :] + p_ref[1, :N, :] + hs1_ref[...]) * dinv_ref[...]
    t = agg + b1_ref[...]
    t = g_ref[...] * t * (1.0 / (1.0 + BN_EPS) ** 0.5) + bt_ref[...]
    scale = 1.0507009873554804934193349852946
    alpha = 1.6732632423543772848170429916717
    t = scale * jnp.where(t > 0, t, alpha * (jnp.exp(t) - 1.0))
    h2 = jnp.dot(t, w2_ref[...], preferred_element_type=jnp.float32)
    hs2_ref[...] = h2 * dinv_ref[...]


def _tc3_body(p_ref, hs2_ref, dinv_ref, b2_ref, out_ref):
    out_ref[...] = ((p_ref[0, :N, :] + p_ref[1, :N, :] + hs2_ref[...])
                    * dinv_ref[...] + b2_ref[...])


_tc1a = pl.pallas_call(
    _tc1a_body,
    out_shape=jax.ShapeDtypeStruct((N, D), jnp.float32),
)

_tc1b = pl.pallas_call(
    _tc1b_body,
    out_shape=[jax.ShapeDtypeStruct((N, D), jnp.float32),
               jax.ShapeDtypeStruct((N, 1), jnp.float32)],
)

_tc2 = pl.pallas_call(
    _tc2_body,
    out_shape=jax.ShapeDtypeStruct((N, D), jnp.float32),
)

_tc3 = pl.pallas_call(
    _tc3_body,
    out_shape=jax.ShapeDtypeStruct((N, D), jnp.float32),
)


def kernel(x, edge_index, W1, b1, gamma1, beta1, W2, b2):
    pad = EPAD - E
    pad_dst = N + (jnp.arange(pad, dtype=jnp.int32) % (NPAD - N))
    pad_src = jnp.arange(pad, dtype=jnp.int32) % N
    src = jnp.concatenate([edge_index[0], pad_src])
    dst = jnp.concatenate([edge_index[1], pad_dst])
    srcp = src.reshape(NT, CHUNKS, K)
    dstp = dst.reshape(NT, CHUNKS, K)
    zeros_rows = jnp.zeros((ROWS_PER_SUB, D), jnp.float32)

    degp = _deg_kernel(dstp)
    degp3 = degp.reshape(NC, DEGPAD, 1)

    h1 = _tc1a(x, W1)
    hs1, dinv = _tc1b(degp3, h1)
    p1 = _agg_kernel(hs1, srcp, dstp, zeros_rows)
    hs2 = _tc2(p1, hs1, dinv, gamma1.reshape(1, D), beta1.reshape(1, D),
               b1.reshape(1, D), W2)
    p2 = _agg_kernel(hs2, srcp, dstp, zeros_rows)
    out = _tc3(p2, hs2, dinv, b2.reshape(1, D))
    return out
